# Initial kernel scaffold; baseline (speedup 1.0000x reference)
import jax
import jax.numpy as jnp
from jax.experimental import pallas as pl


def kernel(x):
    raise NotImplementedError("write your pallas kernel here")


# speedup vs baseline: 17.0731x; 17.0731x over previous
import functools

import jax
import jax.numpy as jnp
from jax import lax
from jax.experimental import pallas as pl
from jax.experimental.pallas import tpu as pltpu
from jax.experimental.pallas import tpu_sc as plsc

_P = 0.12
_B, _S, _C = 4, 4096, 1024
_ROWS = _B * _S

_info = plsc.get_sparse_core_info()
_NC, _NS = _info.num_cores, _info.num_subcores
_NW = _NC * _NS
_RPW = _ROWS // _NW
_K = 64
_NCHUNK = _RPW // _K

_mesh = plsc.VectorSubcoreMesh(core_axis_name="c", subcore_axis_name="s")


@functools.partial(
    pl.kernel,
    mesh=_mesh,
    out_type=jax.ShapeDtypeStruct((_ROWS, _C), jnp.float32),
    scratch_types=[
        pltpu.VMEM((_RPW,), jnp.int32),
        pltpu.VMEM((_K, _C), jnp.float32),
        pltpu.SemaphoreType.DMA,
    ],
)
def _gather_rows(x_hbm, idx_hbm, out_hbm, idx_v, rows_v, sem):
    wid = lax.axis_index("s") * _NC + lax.axis_index("c")
    base = wid * _RPW
    pltpu.sync_copy(idx_hbm.at[pl.ds(base, _RPW)], idx_v)
    for ci in range(_NCHUNK):
        pltpu.async_copy(x_hbm.at[idx_v.at[pl.ds(ci * _K, _K)]], rows_v, sem).wait()
        pltpu.sync_copy(rows_v, out_hbm.at[pl.ds(base + ci * _K, _K)])


def _flat_index():
    prob = jnp.array([_P / 2.0, 1.0 - _P, _P / 2.0], dtype=jnp.float32)
    skey = jax.random.key(42)
    index = jax.random.categorical(skey, jnp.log(prob), shape=(_B, _S)) - 1
    index = index.at[:, 0].set(jnp.clip(index[:, 0], 0, 1))
    index = index.at[:, -1].set(jnp.clip(index[:, -1], -1, 0))
    index = index + jnp.arange(_S, dtype=index.dtype)[None, :]
    index = index + jnp.arange(_B, dtype=index.dtype)[:, None] * _S
    return index.reshape(_ROWS).astype(jnp.int32)


def kernel(x):
    idx = _flat_index()
    out = _gather_rows(x.reshape(_ROWS, _C), idx)
    return out.reshape(_B, _S, _C)
